# baseline (speedup 1.0000x reference)
import functools

import jax
import jax.numpy as jnp
from jax import lax
from jax.experimental import pallas as pl
from jax.experimental.pallas import tpu as pltpu
from jax.experimental.pallas import tpu_sc as plsc

N = 4096
V = 32768
MASK_ID = 32767

NC, NS = 2, 16
NW = NC * NS
RW = N // NW

_sc_mesh = plsc.VectorSubcoreMesh(core_axis_name="c", subcore_axis_name="s")


@functools.partial(
    pl.kernel,
    out_type=jax.ShapeDtypeStruct((N * V,), jnp.float32),
    mesh=_sc_mesh,
    scratch_types=[
        pltpu.VMEM((V,), jnp.float32),
        pltpu.VMEM((RW,), jnp.int32),
        pltpu.VMEM((RW,), jnp.int32),
        pltpu.VMEM((RW,), jnp.float32),
        pltpu.SemaphoreType.DMA,
        pltpu.SemaphoreType.DMA,
    ],
)
def _sc_fill(z_hbm, out_hbm, zero_buf, z_v, idx_v, val_v, sem1, sem2):
    wid = lax.axis_index("s") * NC + lax.axis_index("c")
    base = wid * RW
    pltpu.sync_copy(z_hbm.at[pl.ds(base, RW)], z_v)
    zero_buf[...] = jnp.zeros((V,), jnp.float32)

    lane = lax.iota(jnp.int32, 16)
    for j in range(RW // 16):
        z16 = z_v[pl.ds(16 * j, 16)]
        rows16 = base + 16 * j + lane
        idx_v[pl.ds(16 * j, 16)] = rows16 * V + z16
        val_v[pl.ds(16 * j, 16)] = jnp.where(
            z16 == MASK_ID, jnp.float32(-jnp.inf), jnp.float32(1e9)
        )

    for i in range(RW):
        pltpu.make_async_copy(
            zero_buf, out_hbm.at[pl.ds((base + i) * V, V)], sem1
        ).start()
    for i in range(RW):
        pltpu.make_async_copy(
            zero_buf, out_hbm.at[pl.ds((base + i) * V, V)], sem1
        ).wait()

    pltpu.async_copy(val_v, out_hbm.at[idx_v], sem2).wait()


def _fix_body(z_vmem, z_smem, logits_hbm, out_in_hbm, out_hbm, rowbuf, sem):
    del out_in_hbm
    any_masked = jnp.any(z_vmem[:, :] == MASK_ID)

    @pl.when(any_masked)
    def _():
        col = lax.broadcasted_iota(jnp.int32, (1, 128), 1)

        def body(i, carry):
            zi = z_smem[0, i]

            @pl.when(zi == MASK_ID)
            def _():
                cp = pltpu.make_async_copy(
                    logits_hbm.at[pl.ds(i, 1), :], rowbuf, sem
                )
                cp.start()
                cp.wait()
                tail = rowbuf[:, pl.ds(V - 128, 128)]
                rowbuf[:, pl.ds(V - 128, 128)] = jnp.where(
                    col == 127, jnp.float32(-jnp.inf), tail
                )
                cp2 = pltpu.make_async_copy(
                    rowbuf, out_hbm.at[pl.ds(i, 1), :], sem
                )
                cp2.start()
                cp2.wait()

            return carry

        lax.fori_loop(0, N, body, 0)


def _tc_fixup(out_flat, logits2d, z_t):
    z2 = z_t.reshape(N, 1)
    out2d = out_flat.reshape(N, V)
    return pl.pallas_call(
        _fix_body,
        grid=(1,),
        in_specs=[
            pl.BlockSpec((N, 1), lambda i: (0, 0)),
            pl.BlockSpec((1, N), lambda i: (0, 0), memory_space=pltpu.SMEM),
            pl.BlockSpec(memory_space=pl.ANY),
            pl.BlockSpec(memory_space=pl.ANY),
        ],
        out_specs=pl.BlockSpec(memory_space=pl.ANY),
        out_shape=jax.ShapeDtypeStruct((N, V), jnp.float32),
        scratch_shapes=[
            pltpu.VMEM((1, V), jnp.float32),
            pltpu.SemaphoreType.DMA,
        ],
        input_output_aliases={3: 0},
    )(z2, z_t.reshape(1, N), logits2d, out2d)


def kernel(logits, z_t):
    b, t, v = logits.shape
    lf = logits.reshape(N, V)
    z1 = z_t.reshape(N)
    out_flat = _sc_fill(z1)
    out = _tc_fixup(out_flat, lf, z_t)
    return out.reshape(b, t, v)


# speedup vs baseline: 4.6352x; 4.6352x over previous
import jax
import jax.numpy as jnp
from jax.experimental import pallas as pl
from jax.experimental.pallas import tpu as pltpu

VOCAB = 32768
MASK_ID = 32767
ROWS_PER_BLOCK = 64
CHUNK = 4096


def _blend_kernel(z_ref, logits_hbm, out_ref, scratch, sem):
    i = pl.program_id(0)
    z = z_ref[:, :]
    r, c = out_ref.shape
    any_masked = jnp.any(z == MASK_ID)

    @pl.when(any_masked)
    def _():
        nchunks = c // CHUNK

        def chunk_body(j, carry):
            cp = pltpu.make_async_copy(
                logits_hbm.at[pl.ds(i * r, r), pl.ds(j * CHUNK, CHUNK)],
                scratch,
                sem,
            )
            cp.start()
            cp.wait()
            col = j * CHUNK + jax.lax.broadcasted_iota(
                jnp.int32, (r, CHUNK), 1
            )
            onehot = jnp.where(col == z, jnp.float32(1e9), jnp.float32(0.0))
            lg = jnp.where(
                col == MASK_ID, jnp.float32(-jnp.inf), scratch[:, :]
            )
            out_ref[:, pl.ds(j * CHUNK, CHUNK)] = jnp.where(
                z == MASK_ID, lg, onehot
            )
            return carry

        jax.lax.fori_loop(0, nchunks, chunk_body, 0)

    @pl.when(jnp.logical_not(any_masked))
    def _():
        out_ref[:, :] = jnp.zeros((r, c), jnp.float32)

        col = jax.lax.broadcasted_iota(jnp.int32, (r, c), 1)
        out_ref[:, :] = jnp.where(col == z, jnp.float32(1e9), jnp.float32(0.0))


def kernel(logits, z_t):
    b, t, v = logits.shape
    n = b * t
    lf = logits.reshape(n, v)
    zf = z_t.reshape(n, 1)
    r = ROWS_PER_BLOCK
    out = pl.pallas_call(
        _blend_kernel,
        grid=(n // r,),
        in_specs=[
            pl.BlockSpec((r, 1), lambda i: (i, 0)),
            pl.BlockSpec(memory_space=pl.ANY),
        ],
        out_specs=pl.BlockSpec((r, v), lambda i: (i, 0)),
        out_shape=jax.ShapeDtypeStruct((n, v), jnp.float32),
        scratch_shapes=[
            pltpu.VMEM((r, CHUNK), jnp.float32),
            pltpu.SemaphoreType.DMA,
        ],
    )(zf, lf)
    return out.reshape(b, t, v)
